# baseline (speedup 1.0000x reference)
import functools

import jax
import jax.numpy as jnp
from jax import lax
from jax.experimental import pallas as pl
from jax.experimental.pallas import tpu as pltpu
from jax.experimental.pallas import tpu_sc as plsc

F = 26
V = 100000
D = 16
B = 16384
H = 10
L = 5
C = 2

NW = 32
RPW = B // NW
CB = 64
NCHUNK = RPW // CB
PF = CB * F
GW = 128
NG = PF // GW


def _sc_second_order(idx_flat, xv_flat, tab_flat):
    mesh = plsc.VectorSubcoreMesh(core_axis_name="c", subcore_axis_name="s")

    @functools.partial(
        pl.kernel,
        out_type=jax.ShapeDtypeStruct((B * D,), jnp.float32),
        mesh=mesh,
        compiler_params=pltpu.CompilerParams(use_tc_tiling_on_sc=False),
        scratch_types=[
            pltpu.VMEM((PF,), jnp.int32),
            pltpu.VMEM((PF + 16,), jnp.float32),
            pltpu.VMEM((PF, D), jnp.float32),
            pltpu.VMEM((CB * D,), jnp.float32),
            pltpu.SemaphoreType.DMA,
        ],
    )
    def body(idx_hbm, xv_hbm, tab_hbm, out_hbm, idx_v, xv_v, rows_v, out_v, sem):
        wid = lax.axis_index("s") * 2 + lax.axis_index("c")
        lanes = lax.iota(jnp.int32, 16)

        def chunk(c, carry):
            row0 = wid * RPW + c * CB
            p_base = row0 * F
            pltpu.sync_copy(idx_hbm.at[pl.ds(p_base, PF)], idx_v)
            pltpu.sync_copy(xv_hbm.at[pl.ds(p_base, PF)], xv_v.at[pl.ds(0, PF)])
            copies = [
                pltpu.async_copy(
                    tab_hbm.at[idx_v.at[pl.ds(j * GW, GW)]],
                    rows_v.at[pl.ds(j * GW, GW)],
                    sem,
                )
                for j in range(NG)
            ]
            for cp in copies:
                cp.wait()

            def rowloop(b, carry2):
                p0 = b * F
                xv0 = xv_v[pl.ds(p0, 16)]
                xv1 = xv_v[pl.ds(p0 + 16, 16)]
                acc = jnp.zeros((16,), jnp.float32)
                accsq = jnp.zeros((16,), jnp.float32)
                for f in range(F):
                    src = xv0 if f < 16 else xv1
                    lane = jnp.full((16, 1), f % 16, jnp.int32)
                    xv = lax.gather(
                        src, lane,
                        lax.GatherDimensionNumbers(
                            offset_dims=(), collapsed_slice_dims=(0,),
                            start_index_map=(0,)),
                        slice_sizes=(1,),
                        mode=lax.GatherScatterMode.PROMISE_IN_BOUNDS)
                    e = rows_v[p0 + f]
                    s = e * xv
                    acc = acc + s
                    accsq = accsq + s * s
                out_v[pl.ds(b * D, D)] = 0.5 * (acc * acc - accsq)
                return carry2

            lax.fori_loop(0, CB, rowloop, 0)
            pltpu.sync_copy(out_v, out_hbm.at[pl.ds(row0 * D, CB * D)])
            return carry

        lax.fori_loop(0, NCHUNK, chunk, 0)

    return body(idx_flat, xv_flat, tab_flat)


BT = 512


def _mlp_body(x_ref, w0t_ref, b0_ref, wht_ref, bh_ref, wot_ref, bo_ref, out_ref):
    x = x_ref[...]
    h = jnp.maximum(
        jnp.dot(x, w0t_ref[...], preferred_element_type=jnp.float32)
        + b0_ref[...], 0.0)
    hs = [h]
    for i in range(L - 1):
        h = jnp.maximum(
            jnp.dot(hs[-1], wht_ref[i], preferred_element_type=jnp.float32)
            + bh_ref[i], 0.0)
        hs.append(h)
    for i in range(L):
        out_ref[i] = (
            jnp.dot(hs[i], wot_ref[i], preferred_element_type=jnp.float32)
            + bo_ref[i])


def _mlp(so, w0t, b0, wht, bh, wot, bo):
    grid = (B // BT,)
    return pl.pallas_call(
        _mlp_body,
        grid=grid,
        in_specs=[
            pl.BlockSpec((BT, D), lambda i: (i, 0)),
            pl.BlockSpec((D, H), lambda i: (0, 0)),
            pl.BlockSpec((H,), lambda i: (0,)),
            pl.BlockSpec((L - 1, H, H), lambda i: (0, 0, 0)),
            pl.BlockSpec((L - 1, H), lambda i: (0, 0)),
            pl.BlockSpec((L, H, C), lambda i: (0, 0, 0)),
            pl.BlockSpec((L, C), lambda i: (0, 0)),
        ],
        out_specs=pl.BlockSpec((L, BT, C), lambda i: (0, i, 0)),
        out_shape=jax.ShapeDtypeStruct((L, B, C), jnp.float32),
    )(so, w0t, b0, wht, bh, wot, bo)


def kernel(Xi, Xv, tables, W0, b0, Wh, bh, Wo, bo):
    idx = Xi[:, :, 0].astype(jnp.int32) + (
        jnp.arange(F, dtype=jnp.int32) * V)[None, :]
    idx_flat = idx.reshape(B * F)
    xv_flat = Xv.reshape(B * F)
    tab_flat = tables.reshape(F * V, D)
    so = _sc_second_order(idx_flat, xv_flat, tab_flat).reshape(B, D)
    return _mlp(so, W0.T, b0, Wh.transpose(0, 2, 1), bh,
                Wo.transpose(0, 2, 1), bo)


# speedup vs baseline: 1.2372x; 1.2372x over previous
import functools

import jax
import jax.numpy as jnp
from jax import lax
from jax.experimental import pallas as pl
from jax.experimental.pallas import tpu as pltpu
from jax.experimental.pallas import tpu_sc as plsc

F = 26
V = 100000
D = 16
B = 16384
H = 10
L = 5
C = 2

NW = 32
RPW = B // NW
CB = 64
NCHUNK = RPW // CB
PF = CB * F
GW = 128
NG = PF // GW


def _sc_second_order(idx_flat, xv_flat, tab_flat):
    mesh = plsc.VectorSubcoreMesh(core_axis_name="c", subcore_axis_name="s")

    @functools.partial(
        pl.kernel,
        out_type=jax.ShapeDtypeStruct((B * D,), jnp.float32),
        mesh=mesh,
        compiler_params=pltpu.CompilerParams(use_tc_tiling_on_sc=False),
        scratch_types=[
            pltpu.VMEM((PF,), jnp.int32),
            pltpu.VMEM((PF + 16,), jnp.float32),
            pltpu.VMEM((PF, D), jnp.float32),
            pltpu.VMEM((CB * D,), jnp.float32),
            pltpu.SemaphoreType.DMA,
        ],
    )
    def body(idx_hbm, xv_hbm, tab_hbm, out_hbm, idx_v, xv_v, rows_v, out_v, sem):
        wid = lax.axis_index("s") * 2 + lax.axis_index("c")
        lanes = lax.iota(jnp.int32, 16)

        def chunk(c, carry):
            row0 = wid * RPW + c * CB
            p_base = row0 * F
            pltpu.sync_copy(idx_hbm.at[pl.ds(p_base, PF)], idx_v)
            pltpu.sync_copy(xv_hbm.at[pl.ds(p_base, PF)], xv_v.at[pl.ds(0, PF)])
            copies = [
                pltpu.async_copy(
                    tab_hbm.at[idx_v.at[pl.ds(j * GW, GW)]],
                    rows_v.at[pl.ds(j * GW, GW)],
                    sem,
                )
                for j in range(NG)
            ]
            for cp in copies:
                cp.wait()

            def rowloop(b, carry2):
                p0 = b * F
                xv0 = xv_v[pl.ds(p0, 16)]
                xv1 = xv_v[pl.ds(p0 + 16, 16)]
                acc = jnp.zeros((16,), jnp.float32)
                accsq = jnp.zeros((16,), jnp.float32)
                for f in range(F):
                    src = xv0 if f < 16 else xv1
                    lane = jnp.full((16, 1), f % 16, jnp.int32)
                    xv = lax.gather(
                        src, lane,
                        lax.GatherDimensionNumbers(
                            offset_dims=(), collapsed_slice_dims=(0,),
                            start_index_map=(0,)),
                        slice_sizes=(1,),
                        mode=lax.GatherScatterMode.PROMISE_IN_BOUNDS)
                    e = rows_v[p0 + f]
                    s = e * xv
                    acc = acc + s
                    accsq = accsq + s * s
                out_v[pl.ds(b * D, D)] = 0.5 * (acc * acc - accsq)
                return carry2

            lax.fori_loop(0, CB, rowloop, 0)
            pltpu.sync_copy(out_v, out_hbm.at[pl.ds(row0 * D, CB * D)])
            return carry

        lax.fori_loop(0, NCHUNK, chunk, 0)

    return body(idx_flat, xv_flat, tab_flat)


VP = 102400
PR = VP * D // 128


VB = 12800
PRB = VB // 8


def _pack_body(x_ref, out_ref, z_ref):
    z_ref[...] = jnp.swapaxes(x_ref[0], 0, 1)
    for s in range(8):
        out_ref[:, s * D:(s + 1) * D] = z_ref[pl.Slice(s, PRB, 8), :]


def _pack_table(tab_t):
    return pl.pallas_call(
        _pack_body,
        grid=(F, VP // VB),
        in_specs=[
            pl.BlockSpec((1, D, VB), lambda f, j: (f, 0, j)),
        ],
        out_specs=pl.BlockSpec((PRB, 128), lambda f, j: (f * 8 + j, 0)),
        out_shape=jax.ShapeDtypeStruct((F * PR, 128), jnp.float32),
        scratch_shapes=[pltpu.VMEM((VB, D), jnp.float32)],
    )(tab_t)


BT = 512


def _mlp_body(x_ref, w0t_ref, b0_ref, wht_ref, bh_ref, wot_ref, bo_ref, out_ref):
    x = x_ref[...]
    h = jnp.maximum(
        jnp.dot(x, w0t_ref[...], preferred_element_type=jnp.float32)
        + b0_ref[...], 0.0)
    hs = [h]
    for i in range(L - 1):
        h = jnp.maximum(
            jnp.dot(hs[-1], wht_ref[i], preferred_element_type=jnp.float32)
            + bh_ref[i], 0.0)
        hs.append(h)
    for i in range(L):
        out_ref[i] = (
            jnp.dot(hs[i], wot_ref[i], preferred_element_type=jnp.float32)
            + bo_ref[i])


def _mlp(so, w0t, b0, wht, bh, wot, bo):
    grid = (B // BT,)
    return pl.pallas_call(
        _mlp_body,
        grid=grid,
        in_specs=[
            pl.BlockSpec((BT, D), lambda i: (i, 0)),
            pl.BlockSpec((D, H), lambda i: (0, 0)),
            pl.BlockSpec((H,), lambda i: (0,)),
            pl.BlockSpec((L - 1, H, H), lambda i: (0, 0, 0)),
            pl.BlockSpec((L - 1, H), lambda i: (0, 0)),
            pl.BlockSpec((L, H, C), lambda i: (0, 0, 0)),
            pl.BlockSpec((L, C), lambda i: (0, 0)),
        ],
        out_specs=pl.BlockSpec((L, BT, C), lambda i: (0, i, 0)),
        out_shape=jax.ShapeDtypeStruct((L, B, C), jnp.float32),
    )(so, w0t, b0, wht, bh, wot, bo)


def kernel(Xi, Xv, tables, W0, b0, Wh, bh, Wo, bo):
    idx = Xi[:, :, 0].astype(jnp.int32) + (
        jnp.arange(F, dtype=jnp.int32) * VP)[None, :]
    idx_flat = idx.reshape(B * F)
    xv_flat = Xv.reshape(B * F)
    packed = _pack_table(jnp.transpose(tables, (0, 2, 1)))
    tab_flat = packed.reshape(F * VP, D)
    so = _sc_second_order(idx_flat, xv_flat, tab_flat).reshape(B, D)
    return _mlp(so, W0.T, b0, Wh.transpose(0, 2, 1), bh,
                Wo.transpose(0, 2, 1), bo)


# speedup vs baseline: 1.4909x; 1.2051x over previous
import functools

import jax
import jax.numpy as jnp
from jax import lax
from jax.experimental import pallas as pl
from jax.experimental.pallas import tpu as pltpu
from jax.experimental.pallas import tpu_sc as plsc

F = 26
V = 100000
D = 16
B = 16384
H = 10
L = 5
C = 2

NW = 32
RPW = B // NW
CB = 64
NCHUNK = RPW // CB
PF = CB * F
PE = PF * D
GW = 2048
NG = PE // GW


V2 = 100352


def _compact_body(x_ref, out_ref):
    for d in range(D):
        out_ref[pl.ds(d * V2, V)] = x_ref[0, d]


def _compact_table(tab_t):
    return pl.pallas_call(
        _compact_body,
        grid=(F,),
        in_specs=[pl.BlockSpec((1, D, V), lambda f: (f, 0, 0))],
        out_specs=pl.BlockSpec((D * V2,), lambda f: (f,)),
        out_shape=jax.ShapeDtypeStruct((F * D * V2,), jnp.float32),
    )(tab_t)


def _sc_second_order(idx_flat, xv_flat, tab_flat):
    mesh = plsc.VectorSubcoreMesh(core_axis_name="c", subcore_axis_name="s")

    @functools.partial(
        pl.kernel,
        out_type=jax.ShapeDtypeStruct((B * D,), jnp.float32),
        mesh=mesh,
        compiler_params=pltpu.CompilerParams(use_tc_tiling_on_sc=False),
        scratch_types=[
            pltpu.VMEM((PE,), jnp.int32),
            pltpu.VMEM((PF + 16,), jnp.float32),
            pltpu.VMEM((PE,), jnp.float32),
            pltpu.VMEM((CB * D,), jnp.float32),
            pltpu.SemaphoreType.DMA,
        ],
    )
    def body(idx_hbm, xv_hbm, tab_hbm, out_hbm, idx_v, xv_v, e_v, out_v, sem):
        wid = lax.axis_index("s") * 2 + lax.axis_index("c")

        def chunk(c, carry):
            row0 = wid * RPW + c * CB
            p_base = row0 * F
            pltpu.sync_copy(idx_hbm.at[pl.ds(p_base * D, PE)], idx_v)
            pltpu.sync_copy(xv_hbm.at[pl.ds(p_base, PF)], xv_v.at[pl.ds(0, PF)])
            copies = [
                pltpu.async_copy(
                    tab_hbm.at[idx_v.at[pl.ds(j * GW, GW)]],
                    e_v.at[pl.ds(j * GW, GW)],
                    sem,
                )
                for j in range(NG)
            ]
            for cp in copies:
                cp.wait()

            def rowloop(b, carry2):
                p0 = b * F
                xv0 = xv_v[pl.ds(p0, 16)]
                xv1 = xv_v[pl.ds(p0 + 16, 16)]
                acc = jnp.zeros((16,), jnp.float32)
                accsq = jnp.zeros((16,), jnp.float32)
                for f in range(F):
                    src = xv0 if f < 16 else xv1
                    lane = jnp.full((16, 1), f % 16, jnp.int32)
                    xv = lax.gather(
                        src, lane,
                        lax.GatherDimensionNumbers(
                            offset_dims=(), collapsed_slice_dims=(0,),
                            start_index_map=(0,)),
                        slice_sizes=(1,),
                        mode=lax.GatherScatterMode.PROMISE_IN_BOUNDS)
                    e = e_v[pl.ds((p0 + f) * D, D)]
                    s = e * xv
                    acc = acc + s
                    accsq = accsq + s * s
                out_v[pl.ds(b * D, D)] = 0.5 * (acc * acc - accsq)
                return carry2

            lax.fori_loop(0, CB, rowloop, 0)
            pltpu.sync_copy(out_v, out_hbm.at[pl.ds(row0 * D, CB * D)])
            return carry

        lax.fori_loop(0, NCHUNK, chunk, 0)

    return body(idx_flat, xv_flat, tab_flat)


BT = 512


def _mlp_body(x_ref, w0t_ref, b0_ref, wht_ref, bh_ref, wot_ref, bo_ref, out_ref):
    x = x_ref[...]
    h = jnp.maximum(
        jnp.dot(x, w0t_ref[...], preferred_element_type=jnp.float32)
        + b0_ref[...], 0.0)
    hs = [h]
    for i in range(L - 1):
        h = jnp.maximum(
            jnp.dot(hs[-1], wht_ref[i], preferred_element_type=jnp.float32)
            + bh_ref[i], 0.0)
        hs.append(h)
    for i in range(L):
        out_ref[i] = (
            jnp.dot(hs[i], wot_ref[i], preferred_element_type=jnp.float32)
            + bo_ref[i])


def _mlp(so, w0t, b0, wht, bh, wot, bo):
    grid = (B // BT,)
    return pl.pallas_call(
        _mlp_body,
        grid=grid,
        in_specs=[
            pl.BlockSpec((BT, D), lambda i: (i, 0)),
            pl.BlockSpec((D, H), lambda i: (0, 0)),
            pl.BlockSpec((H,), lambda i: (0,)),
            pl.BlockSpec((L - 1, H, H), lambda i: (0, 0, 0)),
            pl.BlockSpec((L - 1, H), lambda i: (0, 0)),
            pl.BlockSpec((L, H, C), lambda i: (0, 0, 0)),
            pl.BlockSpec((L, C), lambda i: (0, 0)),
        ],
        out_specs=pl.BlockSpec((L, BT, C), lambda i: (0, i, 0)),
        out_shape=jax.ShapeDtypeStruct((L, B, C), jnp.float32),
    )(so, w0t, b0, wht, bh, wot, bo)


def kernel(Xi, Xv, tables, W0, b0, Wh, bh, Wo, bo):
    base = Xi[:, :, 0].astype(jnp.int32) + (
        jnp.arange(F, dtype=jnp.int32) * (D * V2))[None, :]
    idx_all = base[:, :, None] + (
        jnp.arange(D, dtype=jnp.int32) * V2)[None, None, :]
    idx_flat = idx_all.reshape(B * F * D)
    xv_flat = Xv.reshape(B * F)
    tab_flat = _compact_table(jnp.transpose(tables, (0, 2, 1)))
    so = _sc_second_order(idx_flat, xv_flat, tab_flat).reshape(B, D)
    return _mlp(so, W0.T, b0, Wh.transpose(0, 2, 1), bh,
                Wo.transpose(0, 2, 1), bo)


# speedup vs baseline: 2.2342x; 1.4985x over previous
import functools

import jax
import jax.numpy as jnp
from jax import lax
from jax.experimental import pallas as pl
from jax.experimental.pallas import tpu as pltpu
from jax.experimental.pallas import tpu_sc as plsc

F = 26
V = 100000
D = 16
B = 16384
H = 10
L = 5
C = 2

NW = 32
RPW = B // NW
CB = 64
NCHUNK = RPW // CB
PF = CB * F
PE = PF * D
GW = 2048
NG = PE // GW


V2 = 100352


def _compact_body(x_ref, out_ref):
    for d in range(D):
        out_ref[pl.ds(d * V2, V)] = x_ref[0, d]


def _compact_table(tab_t):
    return pl.pallas_call(
        _compact_body,
        grid=(F,),
        in_specs=[pl.BlockSpec((1, D, V), lambda f: (f, 0, 0))],
        out_specs=pl.BlockSpec((D * V2,), lambda f: (f,)),
        out_shape=jax.ShapeDtypeStruct((F * D * V2,), jnp.float32),
    )(tab_t)


def _sc_second_order(idx_flat, xv_flat, tab_flat):
    mesh = plsc.VectorSubcoreMesh(core_axis_name="c", subcore_axis_name="s")

    @functools.partial(
        pl.kernel,
        out_type=jax.ShapeDtypeStruct((B * D,), jnp.float32),
        mesh=mesh,
        compiler_params=pltpu.CompilerParams(use_tc_tiling_on_sc=False),
        scratch_types=[
            pltpu.VMEM((PF,), jnp.int32),
            pltpu.VMEM((PE,), jnp.int32),
            pltpu.VMEM((PF + 16,), jnp.float32),
            pltpu.VMEM((PE,), jnp.float32),
            pltpu.VMEM((CB * D,), jnp.float32),
            pltpu.SemaphoreType.DMA,
        ],
    )
    def body(idx_hbm, xv_hbm, tab_hbm, out_hbm, base_v, idx_v, xv_v, e_v,
             out_v, sem):
        wid = lax.axis_index("s") * 2 + lax.axis_index("c")
        vstep = lax.iota(jnp.int32, 16) * V2
        gdn = lax.GatherDimensionNumbers(
            offset_dims=(), collapsed_slice_dims=(0,), start_index_map=(0,))

        def chunk(c, carry):
            row0 = wid * RPW + c * CB
            p_base = row0 * F
            pltpu.sync_copy(idx_hbm.at[pl.ds(p_base, PF)], base_v)
            pltpu.sync_copy(xv_hbm.at[pl.ds(p_base, PF)], xv_v.at[pl.ds(0, PF)])

            def bgroup(g, carry3):
                bb = base_v[pl.ds(g * 16, 16)]
                for i in range(16):
                    lane_i = jnp.full((16, 1), i, jnp.int32)
                    bcast = lax.gather(
                        bb, lane_i, gdn, slice_sizes=(1,),
                        mode=lax.GatherScatterMode.PROMISE_IN_BOUNDS)
                    idx_v[pl.ds((g * 16 + i) * D, D)] = bcast + vstep
                return carry3

            lax.fori_loop(0, PF // 16, bgroup, 0)
            copies = [
                pltpu.async_copy(
                    tab_hbm.at[idx_v.at[pl.ds(j * GW, GW)]],
                    e_v.at[pl.ds(j * GW, GW)],
                    sem,
                )
                for j in range(NG)
            ]
            for cp in copies:
                cp.wait()

            def rowloop(b, carry2):
                p0 = b * F
                xv0 = xv_v[pl.ds(p0, 16)]
                xv1 = xv_v[pl.ds(p0 + 16, 16)]
                acc = jnp.zeros((16,), jnp.float32)
                accsq = jnp.zeros((16,), jnp.float32)
                for f in range(F):
                    src = xv0 if f < 16 else xv1
                    lane = jnp.full((16, 1), f % 16, jnp.int32)
                    xv = lax.gather(
                        src, lane,
                        lax.GatherDimensionNumbers(
                            offset_dims=(), collapsed_slice_dims=(0,),
                            start_index_map=(0,)),
                        slice_sizes=(1,),
                        mode=lax.GatherScatterMode.PROMISE_IN_BOUNDS)
                    e = e_v[pl.ds((p0 + f) * D, D)]
                    s = e * xv
                    acc = acc + s
                    accsq = accsq + s * s
                out_v[pl.ds(b * D, D)] = 0.5 * (acc * acc - accsq)
                return carry2

            lax.fori_loop(0, CB, rowloop, 0)
            pltpu.sync_copy(out_v, out_hbm.at[pl.ds(row0 * D, CB * D)])
            return carry

        lax.fori_loop(0, NCHUNK, chunk, 0)

    return body(idx_flat, xv_flat, tab_flat)


BT = 512


def _mlp_body(x_ref, w0t_ref, b0_ref, wht_ref, bh_ref, wot_ref, bo_ref, out_ref):
    x = x_ref[...]
    h = jnp.maximum(
        jnp.dot(x, w0t_ref[...], preferred_element_type=jnp.float32)
        + b0_ref[...], 0.0)
    hs = [h]
    for i in range(L - 1):
        h = jnp.maximum(
            jnp.dot(hs[-1], wht_ref[i], preferred_element_type=jnp.float32)
            + bh_ref[i], 0.0)
        hs.append(h)
    for i in range(L):
        out_ref[i] = (
            jnp.dot(hs[i], wot_ref[i], preferred_element_type=jnp.float32)
            + bo_ref[i])


def _mlp(so, w0t, b0, wht, bh, wot, bo):
    grid = (B // BT,)
    return pl.pallas_call(
        _mlp_body,
        grid=grid,
        in_specs=[
            pl.BlockSpec((BT, D), lambda i: (i, 0)),
            pl.BlockSpec((D, H), lambda i: (0, 0)),
            pl.BlockSpec((H,), lambda i: (0,)),
            pl.BlockSpec((L - 1, H, H), lambda i: (0, 0, 0)),
            pl.BlockSpec((L - 1, H), lambda i: (0, 0)),
            pl.BlockSpec((L, H, C), lambda i: (0, 0, 0)),
            pl.BlockSpec((L, C), lambda i: (0, 0)),
        ],
        out_specs=pl.BlockSpec((L, BT, C), lambda i: (0, i, 0)),
        out_shape=jax.ShapeDtypeStruct((L, B, C), jnp.float32),
    )(so, w0t, b0, wht, bh, wot, bo)


def kernel(Xi, Xv, tables, W0, b0, Wh, bh, Wo, bo):
    base = Xi[:, :, 0].astype(jnp.int32) + (
        jnp.arange(F, dtype=jnp.int32) * (D * V2))[None, :]
    idx_flat = base.reshape(B * F)
    xv_flat = Xv.reshape(B * F)
    tab_flat = _compact_table(jnp.transpose(tables, (0, 2, 1)))
    so = _sc_second_order(idx_flat, xv_flat, tab_flat).reshape(B, D)
    return _mlp(so, W0.T, b0, Wh.transpose(0, 2, 1), bh,
                Wo.transpose(0, 2, 1), bo)


# speedup vs baseline: 2.2658x; 1.0142x over previous
import functools

import jax
import jax.numpy as jnp
from jax import lax
from jax.experimental import pallas as pl
from jax.experimental.pallas import tpu as pltpu
from jax.experimental.pallas import tpu_sc as plsc

F = 26
V = 100000
D = 16
B = 16384
H = 10
L = 5
C = 2

NW = 32
RPW = B // NW
CB = 128
NCHUNK = RPW // CB
PF = CB * F
PE = PF * D
GW = 2048
NG = PE // GW


V2 = 100352


def _compact_body(x_ref, out_ref):
    for d in range(D):
        out_ref[pl.ds(d * V2, V)] = x_ref[0, d]


def _compact_table(tab_t):
    return pl.pallas_call(
        _compact_body,
        grid=(F,),
        in_specs=[pl.BlockSpec((1, D, V), lambda f: (f, 0, 0))],
        out_specs=pl.BlockSpec((D * V2,), lambda f: (f,)),
        out_shape=jax.ShapeDtypeStruct((F * D * V2,), jnp.float32),
    )(tab_t)


def _sc_second_order(idx_flat, xv_flat, tab_flat):
    mesh = plsc.VectorSubcoreMesh(core_axis_name="c", subcore_axis_name="s")

    @functools.partial(
        pl.kernel,
        out_type=jax.ShapeDtypeStruct((B * D,), jnp.float32),
        mesh=mesh,
        compiler_params=pltpu.CompilerParams(use_tc_tiling_on_sc=False),
        scratch_types=[
            pltpu.VMEM((PF,), jnp.int32),
            pltpu.VMEM((PE,), jnp.int32),
            pltpu.VMEM((PF + 16,), jnp.float32),
            pltpu.VMEM((PE,), jnp.float32),
            pltpu.VMEM((CB * D,), jnp.float32),
            pltpu.SemaphoreType.DMA,
        ],
    )
    def body(idx_hbm, xv_hbm, tab_hbm, out_hbm, base_v, idx_v, xv_v, e_v,
             out_v, sem):
        wid = lax.axis_index("s") * 2 + lax.axis_index("c")
        vstep = lax.iota(jnp.int32, 16) * V2
        gdn = lax.GatherDimensionNumbers(
            offset_dims=(), collapsed_slice_dims=(0,), start_index_map=(0,))

        def chunk(c, carry):
            row0 = wid * RPW + c * CB
            p_base = row0 * F
            pltpu.sync_copy(idx_hbm.at[pl.ds(p_base, PF)], base_v)
            pltpu.sync_copy(xv_hbm.at[pl.ds(p_base, PF)], xv_v.at[pl.ds(0, PF)])

            def bgroup(g, carry3):
                bb = base_v[pl.ds(g * 16, 16)]
                for i in range(16):
                    lane_i = jnp.full((16, 1), i, jnp.int32)
                    bcast = lax.gather(
                        bb, lane_i, gdn, slice_sizes=(1,),
                        mode=lax.GatherScatterMode.PROMISE_IN_BOUNDS)
                    idx_v[pl.ds((g * 16 + i) * D, D)] = bcast + vstep
                return carry3

            lax.fori_loop(0, PF // 16, bgroup, 0)
            copies = [
                pltpu.async_copy(
                    tab_hbm.at[idx_v.at[pl.ds(j * GW, GW)]],
                    e_v.at[pl.ds(j * GW, GW)],
                    sem,
                )
                for j in range(NG)
            ]
            for cp in copies:
                cp.wait()

            def rowloop(b, carry2):
                p0 = b * F
                xv0 = xv_v[pl.ds(p0, 16)]
                xv1 = xv_v[pl.ds(p0 + 16, 16)]
                acc = jnp.zeros((16,), jnp.float32)
                accsq = jnp.zeros((16,), jnp.float32)
                for f in range(F):
                    src = xv0 if f < 16 else xv1
                    lane = jnp.full((16, 1), f % 16, jnp.int32)
                    xv = lax.gather(
                        src, lane,
                        lax.GatherDimensionNumbers(
                            offset_dims=(), collapsed_slice_dims=(0,),
                            start_index_map=(0,)),
                        slice_sizes=(1,),
                        mode=lax.GatherScatterMode.PROMISE_IN_BOUNDS)
                    e = e_v[pl.ds((p0 + f) * D, D)]
                    s = e * xv
                    acc = acc + s
                    accsq = accsq + s * s
                out_v[pl.ds(b * D, D)] = 0.5 * (acc * acc - accsq)
                return carry2

            lax.fori_loop(0, CB, rowloop, 0)
            pltpu.sync_copy(out_v, out_hbm.at[pl.ds(row0 * D, CB * D)])
            return carry

        lax.fori_loop(0, NCHUNK, chunk, 0)

    return body(idx_flat, xv_flat, tab_flat)


BT = 512


def _mlp_body(x_ref, w0t_ref, b0_ref, wht_ref, bh_ref, wot_ref, bo_ref, out_ref):
    x = x_ref[...]
    h = jnp.maximum(
        jnp.dot(x, w0t_ref[...], preferred_element_type=jnp.float32)
        + b0_ref[...], 0.0)
    hs = [h]
    for i in range(L - 1):
        h = jnp.maximum(
            jnp.dot(hs[-1], wht_ref[i], preferred_element_type=jnp.float32)
            + bh_ref[i], 0.0)
        hs.append(h)
    for i in range(L):
        out_ref[i] = (
            jnp.dot(hs[i], wot_ref[i], preferred_element_type=jnp.float32)
            + bo_ref[i])


def _mlp(so, w0t, b0, wht, bh, wot, bo):
    grid = (B // BT,)
    return pl.pallas_call(
        _mlp_body,
        grid=grid,
        in_specs=[
            pl.BlockSpec((BT, D), lambda i: (i, 0)),
            pl.BlockSpec((D, H), lambda i: (0, 0)),
            pl.BlockSpec((H,), lambda i: (0,)),
            pl.BlockSpec((L - 1, H, H), lambda i: (0, 0, 0)),
            pl.BlockSpec((L - 1, H), lambda i: (0, 0)),
            pl.BlockSpec((L, H, C), lambda i: (0, 0, 0)),
            pl.BlockSpec((L, C), lambda i: (0, 0)),
        ],
        out_specs=pl.BlockSpec((L, BT, C), lambda i: (0, i, 0)),
        out_shape=jax.ShapeDtypeStruct((L, B, C), jnp.float32),
    )(so, w0t, b0, wht, bh, wot, bo)


def kernel(Xi, Xv, tables, W0, b0, Wh, bh, Wo, bo):
    base = Xi[:, :, 0].astype(jnp.int32) + (
        jnp.arange(F, dtype=jnp.int32) * (D * V2))[None, :]
    idx_flat = base.reshape(B * F)
    xv_flat = Xv.reshape(B * F)
    tab_flat = _compact_table(jnp.transpose(tables, (0, 2, 1)))
    so = _sc_second_order(idx_flat, xv_flat, tab_flat).reshape(B, D)
    return _mlp(so, W0.T, b0, Wh.transpose(0, 2, 1), bh,
                Wo.transpose(0, 2, 1), bo)
